# baseline (speedup 1.0000x reference)
import jax
import jax.numpy as jnp
from jax.experimental import pallas as pl
from jax.experimental.pallas import tpu as pltpu

N, E = 1_000_000, 256
BLOCK = 20_000


def _body(x_ref, w_ref, o_ref):
    wr = w_ref[...]
    x_lo = x_ref[:, :128]
    x_hi = x_ref[:, 128:]
    dims = (((1,), (1,)), ((), ()))
    s = (jax.lax.dot_general(wr[:, :128], x_lo, dims,
                             preferred_element_type=jnp.float32)
         + jax.lax.dot_general(wr[:, 128:], x_hi, dims,
                               preferred_element_type=jnp.float32))
    num = (jax.lax.dot_general(wr[:, :128], x_lo * x_lo, dims,
                               preferred_element_type=jnp.float32)
           + jax.lax.dot_general(wr[:, 128:], x_hi * x_hi, dims,
                                 preferred_element_type=jnp.float32))
    o_ref[...] = jnp.where(s == 0.0, 0.0, num / s)[None]


def kernel(x, w):
    grid = (N // BLOCK,)
    out = pl.pallas_call(
        _body,
        grid=grid,
        in_specs=[
            pl.BlockSpec((BLOCK, E), lambda i: (i, 0)),
            pl.BlockSpec((1, E), lambda i: (0, 0)),
        ],
        out_specs=pl.BlockSpec((1, 1, BLOCK), lambda i: (i, 0, 0)),
        out_shape=jax.ShapeDtypeStruct((N // BLOCK, 1, BLOCK), jnp.float32),
        compiler_params=pltpu.CompilerParams(
            dimension_semantics=("parallel",),
        ),
    )(x, w.reshape(1, E))
    return out.reshape(N)


# speedup vs baseline: 1.1418x; 1.1418x over previous
import jax
import jax.numpy as jnp
from jax.experimental import pallas as pl
from jax.experimental.pallas import tpu as pltpu

N, E = 1_000_000, 256
BLOCK = 22_528


def _body(x_ref, w_ref, o_ref):
    wr = w_ref[...]
    x_lo = x_ref[:, :128]
    x_hi = x_ref[:, 128:]
    dims = (((1,), (1,)), ((), ()))
    s = (jax.lax.dot_general(wr[:, :128], x_lo, dims,
                             preferred_element_type=jnp.float32)
         + jax.lax.dot_general(wr[:, 128:], x_hi, dims,
                               preferred_element_type=jnp.float32))
    num = (jax.lax.dot_general(wr[:, :128], x_lo * x_lo, dims,
                               preferred_element_type=jnp.float32)
           + jax.lax.dot_general(wr[:, 128:], x_hi * x_hi, dims,
                                 preferred_element_type=jnp.float32))
    o_ref[...] = jnp.where(s == 0.0, 0.0, num / s).reshape(BLOCK)


def kernel(x, w):
    grid = (pl.cdiv(N, BLOCK),)
    out = pl.pallas_call(
        _body,
        grid=grid,
        in_specs=[
            pl.BlockSpec((BLOCK, E), lambda i: (i, 0)),
            pl.BlockSpec((1, E), lambda i: (0, 0)),
        ],
        out_specs=pl.BlockSpec((BLOCK,), lambda i: (i,)),
        out_shape=jax.ShapeDtypeStruct((N,), jnp.float32),
        compiler_params=pltpu.CompilerParams(
            dimension_semantics=("parallel",),
        ),
    )(x, w.reshape(1, E))
    return out
